# Initial kernel scaffold; baseline (speedup 1.0000x reference)
import jax
import jax.numpy as jnp
from jax.experimental import pallas as pl


def kernel(input_ids, wte, wpe):
    raise NotImplementedError("write your pallas kernel here")


# speedup vs baseline: 1.0598x; 1.0598x over previous
import functools

import jax
import jax.numpy as jnp
from jax import lax
from jax.experimental import pallas as pl
from jax.experimental.pallas import tpu as pltpu
from jax.experimental.pallas import tpu_sc as plsc

B, T, D = 4, 2048, 768
VOCAB = 50257
NC, NS, L = 2, 16, 16
NW = NC * NS
TPW = T // NW
CS = 32
NCPB = TPW // CS
NCHUNK = B * NCPB
NBUF = 2

_mesh = plsc.VectorSubcoreMesh(
    core_axis_name="c", subcore_axis_name="s", num_cores=NC, num_subcores=NS
)


@functools.partial(
    pl.kernel,
    out_type=jax.ShapeDtypeStruct((B, T, D), jnp.float32),
    mesh=_mesh,
    scratch_types=[
        pltpu.VMEM((NCHUNK, CS), jnp.int32),
        pltpu.VMEM((TPW, D), jnp.float32),
        [pltpu.VMEM((CS, D), jnp.float32) for _ in range(NBUF)],
        pltpu.SemaphoreType.DMA,
        pltpu.SemaphoreType.DMA,
        [pltpu.SemaphoreType.DMA for _ in range(NBUF)],
        [pltpu.SemaphoreType.DMA for _ in range(NBUF)],
    ],
)
def _emb_lookup(ids_hbm, wte_hbm, wpe_hbm, out_hbm,
                idx_v, wpe_v, rows_v, isem, wsem, gsems, ssems):
    wid = lax.axis_index("s") * NC + lax.axis_index("c")
    t0 = wid * TPW

    idx_descs = []
    for k in range(NCHUNK):
        b, c = k // NCPB, k % NCPB
        idx_descs.append(
            pltpu.async_copy(
                ids_hbm.at[b, pl.ds(t0 + c * CS, CS)], idx_v.at[k], isem
            )
        )
    wpe_desc = pltpu.async_copy(wpe_hbm.at[pl.ds(t0, TPW), :], wpe_v, wsem)
    for d in idx_descs:
        d.wait()

    def start_gather(k, p):
        return pltpu.async_copy(wte_hbm.at[idx_v.at[k]], rows_v[p], gsems[p])

    def start_store(k, p):
        b, c = k // NCPB, k % NCPB
        return pltpu.async_copy(
            rows_v[p], out_hbm.at[b, pl.ds(t0 + c * CS, CS), :], ssems[p]
        )

    def add_wpe(k, p):
        c = k % NCPB
        rp = rows_v[p]

        def body(i, _):
            for j in range(D // L):
                sl = pl.ds(j * L, L)
                plsc.addupdate(rp.at[i, sl], wpe_v[c * CS + i, sl])
            return _

        lax.fori_loop(0, CS, body, None)

    g_descs = {}
    s_descs = {}
    for k in range(NBUF):
        g_descs[k] = start_gather(k, k)
    wpe_desc.wait()
    for k in range(NCHUNK):
        p = k % NBUF
        g_descs[k].wait()
        add_wpe(k, p)
        s_descs[k] = start_store(k, p)
        if k + NBUF < NCHUNK:
            s_descs[k].wait()
            g_descs[k + NBUF] = start_gather(k + NBUF, p)
    for k in range(NCHUNK - NBUF, NCHUNK):
        s_descs[k].wait()


def kernel(input_ids, wte, wpe):
    ids32 = input_ids.astype(jnp.int32)
    return _emb_lookup(ids32, wte, wpe)


# speedup vs baseline: 1.2892x; 1.2164x over previous
import functools

import jax
import jax.numpy as jnp
from jax import lax
from jax.experimental import pallas as pl
from jax.experimental.pallas import tpu as pltpu
from jax.experimental.pallas import tpu_sc as plsc

B, T, D = 4, 2048, 768
VOCAB = 50257
NC, NS, L = 2, 16, 16
NW = NC * NS
TPW = T // NW
CS = 32
NCPB = TPW // CS
NCHUNK = B * NCPB
NBUF = 3

_mesh = plsc.VectorSubcoreMesh(
    core_axis_name="c", subcore_axis_name="s", num_cores=NC, num_subcores=NS
)


@functools.partial(
    pl.kernel,
    out_type=jax.ShapeDtypeStruct((B, T, D), jnp.float32),
    mesh=_mesh,
    scratch_types=[
        pltpu.VMEM((NCHUNK, CS), jnp.int32),
        pltpu.VMEM((TPW, D), jnp.float32),
        [pltpu.VMEM((CS, D), jnp.float32) for _ in range(NBUF)],
        pltpu.SemaphoreType.DMA,
        pltpu.SemaphoreType.DMA,
        [pltpu.SemaphoreType.DMA for _ in range(NBUF)],
        [pltpu.SemaphoreType.DMA for _ in range(NBUF)],
    ],
)
def _emb_lookup(ids_hbm, wte_hbm, wpe_hbm, out_hbm,
                idx_v, wpe_v, rows_v, isem, wsem, gsems, ssems):
    wid = lax.axis_index("s") * NC + lax.axis_index("c")
    t0 = wid * TPW

    idx_descs = []
    for k in range(NCHUNK):
        b, c = k // NCPB, k % NCPB
        idx_descs.append(
            pltpu.async_copy(
                ids_hbm.at[b, pl.ds(t0 + c * CS, CS)], idx_v.at[k], isem
            )
        )
    wpe_desc = pltpu.async_copy(wpe_hbm.at[pl.ds(t0, TPW), :], wpe_v, wsem)
    for d in idx_descs:
        d.wait()

    def start_gather(k, p):
        return pltpu.async_copy(wte_hbm.at[idx_v.at[k]], rows_v[p], gsems[p])

    def start_store(k, p):
        b, c = k // NCPB, k % NCPB
        return pltpu.async_copy(
            rows_v[p], out_hbm.at[b, pl.ds(t0 + c * CS, CS), :], ssems[p]
        )

    def add_wpe(k, p):
        c = k % NCPB
        rp = rows_v[p]

        @plsc.parallel_loop(0, CS, unroll=2)
        def _(i):
            for j in range(D // L):
                sl = pl.ds(j * L, L)
                plsc.addupdate(rp.at[i, sl], wpe_v[c * CS + i, sl])

    g_descs = {}
    s_descs = {}
    for k in range(NBUF):
        g_descs[k] = start_gather(k, k)
    wpe_desc.wait()
    for k in range(NCHUNK):
        p = k % NBUF
        g_descs[k].wait()
        add_wpe(k, p)
        s_descs[k] = start_store(k, p)
        j = k + NBUF - 1
        if 1 <= k and j < NCHUNK:
            s_descs[k - 1].wait()
            g_descs[j] = start_gather(j, j % NBUF)
    for k in range(NCHUNK - NBUF, NCHUNK):
        s_descs[k].wait()


def kernel(input_ids, wte, wpe):
    ids32 = input_ids.astype(jnp.int32)
    return _emb_lookup(ids32, wte, wpe)
